# Initial kernel scaffold; baseline (speedup 1.0000x reference)
import jax
import jax.numpy as jnp
from jax.experimental import pallas as pl


def kernel(x, edge_index, edge_weight, W1, b1, W2, b2):
    raise NotImplementedError("write your pallas kernel here")


# speedup vs baseline: 92.8919x; 92.8919x over previous
import functools

import jax
import jax.numpy as jnp
from jax import lax
from jax.experimental import pallas as pl
from jax.experimental.pallas import tpu as pltpu
from jax.experimental.pallas import tpu_sc as plsc

NC = 2
NS = 16
NW = NC * NS
L = 16
GRP = 128
CHUNK = 512
NGRP = CHUNK // GRP
MUL_UNROLL = 16


def _tile_rows(n):
  sl = -(-n // NS)
  sl = -(-sl // 8) * 8
  return sl, n - (NS - 1) * sl


def _sizes(n, chunk=CHUNK):
  out = [chunk] * (n // chunk)
  if n % chunk:
    out.append(n % chunk)
  return out


CHUNK_D = 2048
NGRP_D = CHUNK_D // GRP


def _make_deg_pass(E, N):
  nchunks = E // CHUNK_D
  base, rem = divmod(nchunks, NW)
  nstep = -(-nchunks // NW)
  SL, SL_LAST = _tile_rows(N)
  mesh = plsc.VectorSubcoreMesh(core_axis_name="c", subcore_axis_name="s")

  @functools.partial(
      pl.kernel,
      out_type=jax.ShapeDtypeStruct((NC * N,), jnp.float32),
      mesh=mesh,
      scratch_types=[
          pltpu.VMEM_SHARED((N,), jnp.float32),
          pltpu.VMEM((2, NGRP_D, GRP), jnp.int32),
          pltpu.VMEM((2, CHUNK_D), jnp.float32),
          pltpu.VMEM((CHUNK_D,), jnp.float32),
          pltpu.SemaphoreType.DMA,
          pltpu.SemaphoreType.DMA,
          pltpu.SemaphoreType.DMA,
          pltpu.SemaphoreType.DMA,
      ],
  )
  def deg_pass(dst_hbm, ew_hbm, out_hbm, deg_sh, dstv, eww, stagebuf,
               si0, si1, ss0, ss1):
    c = lax.axis_index("c")
    s = lax.axis_index("s")
    wid = s * NC + c
    sem_i, sem_s = [si0, si1], [ss0, ss1]
    stage = stagebuf

    def zf(i, carry):
      stage[pl.ds(i * L, L)] = jnp.zeros((L,), jnp.float32)
      return carry
    lax.fori_loop(0, CHUNK_D // L, zf, 0)

    @pl.when(s < NS - 1)
    def _():
      off = 0
      for sz in _sizes(SL, CHUNK_D):
        pltpu.sync_copy(stage.at[pl.ds(0, sz)], deg_sh.at[pl.ds(s * SL + off, sz)])
        off += sz

    @pl.when(s == NS - 1)
    def _():
      off = (NS - 1) * SL
      for sz in _sizes(SL_LAST, CHUNK_D):
        pltpu.sync_copy(stage.at[pl.ds(0, sz)], deg_sh.at[pl.ds(off, sz)])
        off += sz

    plsc.subcore_barrier()

    n_i = base + jnp.where(wid < rem, 1, 0)

    def idx_issue(j, b):
      @pl.when(j < n_i)
      def _():
        cid = wid + NW * j
        pltpu.async_copy(dst_hbm.at[pl.ds(cid * NGRP_D, NGRP_D)], dstv.at[b],
                         sem_i[b])
        pltpu.async_copy(ew_hbm.at[pl.ds(cid * CHUNK_D, CHUNK_D)], eww.at[b],
                         sem_i[b])

    def scatter_issue(j, b):
      @pl.when(j < n_i)
      def _():
        cid = wid + NW * j
        pltpu.make_async_copy(dst_hbm.at[pl.ds(cid * NGRP_D, NGRP_D)],
                              dstv.at[b], sem_i[b]).wait()
        pltpu.make_async_copy(ew_hbm.at[pl.ds(cid * CHUNK_D, CHUNK_D)],
                              eww.at[b], sem_i[b]).wait()
        for g in range(NGRP_D):
          pltpu.async_copy(eww.at[b, pl.ds(g * GRP, GRP)],
                           deg_sh.at[dstv.at[b, g]], sem_s[b], add=True)

    def scatter_drain(j, b):
      @pl.when(j < n_i)
      def _():
        for g in range(NGRP_D):
          pltpu.make_async_copy(eww.at[b, pl.ds(g * GRP, GRP)],
                                deg_sh.at[dstv.at[b, g]], sem_s[b]).wait()

    idx_issue(0, 0)
    idx_issue(1, 1)

    def pipe_body(q, carry):
      j0 = 2 * q
      scatter_issue(j0, 0)
      scatter_drain(j0, 0)
      idx_issue(j0 + 2, 0)
      scatter_issue(j0 + 1, 1)
      scatter_drain(j0 + 1, 1)
      idx_issue(j0 + 3, 1)
      return carry
    lax.fori_loop(0, (nstep + 1) // 2, pipe_body, 0)

    plsc.subcore_barrier()

    @pl.when(s < NS - 1)
    def _():
      off = 0
      for sz in _sizes(SL, CHUNK_D):
        pltpu.sync_copy(deg_sh.at[pl.ds(s * SL + off, sz)], stage.at[pl.ds(0, sz)])
        pltpu.sync_copy(stage.at[pl.ds(0, sz)],
                        out_hbm.at[pl.ds(c * N + s * SL + off, sz)])
        off += sz

    @pl.when(s == NS - 1)
    def _():
      off = (NS - 1) * SL
      for sz in _sizes(SL_LAST, CHUNK_D):
        pltpu.sync_copy(deg_sh.at[pl.ds(off, sz)], stage.at[pl.ds(0, sz)])
        pltpu.sync_copy(stage.at[pl.ds(0, sz)],
                        out_hbm.at[pl.ds(c * N + off, sz)])
        off += sz

  return deg_pass


def _make_edge_pass(E, N):
  nchunks = E // CHUNK
  base, rem = divmod(nchunks, NW)
  nstep = -(-nchunks // NW)
  SL, SL_LAST = _tile_rows(N)
  mesh = plsc.VectorSubcoreMesh(core_axis_name="c", subcore_axis_name="s")

  @functools.partial(
      pl.kernel,
      out_type=jax.ShapeDtypeStruct((NC * N, L), jnp.float32),
      mesh=mesh,
      compiler_params=pltpu.CompilerParams(use_tc_tiling_on_sc=False),
      scratch_types=[
          pltpu.VMEM_SHARED((N, L), jnp.float32),
          pltpu.VMEM((2, NGRP, GRP), jnp.int32),
          pltpu.VMEM((2, NGRP, GRP), jnp.int32),
          pltpu.VMEM((2, CHUNK), jnp.float32),
          pltpu.VMEM((2, CHUNK, L), jnp.float32),
          pltpu.SemaphoreType.DMA,
          pltpu.SemaphoreType.DMA,
          pltpu.SemaphoreType.DMA,
          pltpu.SemaphoreType.DMA,
          pltpu.SemaphoreType.DMA,
          pltpu.SemaphoreType.DMA,
      ],
  )
  def edge_pass(src_hbm, dst_hbm, ew_hbm, table_hbm, out_hbm,
                acc, srcv, dstv, eww, rows, si0, si1, sg0, sg1, ss0, ss1):
    c = lax.axis_index("c")
    s = lax.axis_index("s")
    wid = s * NC + c
    sem_i, sem_g, sem_s = [si0, si1], [sg0, sg1], [ss0, ss1]
    stage = rows.at[0]

    def zf(i, carry):
      stage[i, :] = jnp.zeros((L,), jnp.float32)
      return carry
    lax.fori_loop(0, CHUNK, zf, 0)

    @pl.when(s < NS - 1)
    def _():
      off = 0
      for sz in _sizes(SL):
        pltpu.sync_copy(stage.at[pl.ds(0, sz)], acc.at[pl.ds(s * SL + off, sz)])
        off += sz

    @pl.when(s == NS - 1)
    def _():
      off = (NS - 1) * SL
      for sz in _sizes(SL_LAST):
        pltpu.sync_copy(stage.at[pl.ds(0, sz)], acc.at[pl.ds(off, sz)])
        off += sz

    plsc.subcore_barrier()

    n_i = base + jnp.where(wid < rem, 1, 0)

    def _idx_copies(j, b):
      cid = wid + NW * j
      return [
          (src_hbm.at[pl.ds(cid * NGRP, NGRP)], srcv.at[b]),
          (dst_hbm.at[pl.ds(cid * NGRP, NGRP)], dstv.at[b]),
          (ew_hbm.at[pl.ds(cid * CHUNK, CHUNK)], eww.at[b]),
      ]

    def idx_issue(j, b):
      @pl.when(j < n_i)
      def _():
        for src, dst in _idx_copies(j, b):
          pltpu.async_copy(src, dst, sem_i[b])

    def gather_issue(j, b):
      @pl.when(j < n_i)
      def _():
        for src, dst in _idx_copies(j, b):
          pltpu.make_async_copy(src, dst, sem_i[b]).wait()
        for g in range(NGRP):
          pltpu.async_copy(table_hbm.at[srcv.at[b, g]],
                           rows.at[b, pl.ds(g * GRP, GRP)], sem_g[b])

    def mul_scatter(j, b):
      @pl.when(j < n_i)
      def _():
        for g in range(NGRP):
          pltpu.make_async_copy(table_hbm.at[srcv.at[b, g]],
                                rows.at[b, pl.ds(g * GRP, GRP)],
                                sem_g[b]).wait()
        rb = rows.at[b]
        eb = eww.at[b]

        def mul_body(g, carry2):
          ew16 = eb[pl.ds(g * L, L)]
          for i in range(L):
            e = g * L + i
            rb[e, :] = rb[e, :] * ew16[i]
          return carry2
        lax.fori_loop(0, CHUNK // L, mul_body, 0)

        for g in range(NGRP):
          pltpu.async_copy(rows.at[b, pl.ds(g * GRP, GRP)],
                           acc.at[dstv.at[b, g]], sem_s[b], add=True)

    def scatter_drain(j, b):
      @pl.when(j < n_i)
      def _():
        for g in range(NGRP):
          pltpu.make_async_copy(rows.at[b, pl.ds(g * GRP, GRP)],
                                acc.at[dstv.at[b, g]], sem_s[b]).wait()

    idx_issue(0, 0)
    idx_issue(1, 1)
    gather_issue(0, 0)

    def pipe_body(q, carry):
      j0 = 2 * q
      gather_issue(j0 + 1, 1)
      mul_scatter(j0, 0)
      scatter_drain(j0, 0)
      idx_issue(j0 + 2, 0)
      gather_issue(j0 + 2, 0)
      mul_scatter(j0 + 1, 1)
      scatter_drain(j0 + 1, 1)
      idx_issue(j0 + 3, 1)
      return carry
    lax.fori_loop(0, (nstep + 1) // 2, pipe_body, 0)

    plsc.subcore_barrier()

    @pl.when(s < NS - 1)
    def _():
      off = 0
      for sz in _sizes(SL):
        pltpu.sync_copy(acc.at[pl.ds(s * SL + off, sz)], stage.at[pl.ds(0, sz)])
        pltpu.sync_copy(stage.at[pl.ds(0, sz)],
                        out_hbm.at[pl.ds(c * N + s * SL + off, sz)])
        off += sz

    @pl.when(s == NS - 1)
    def _():
      off = (NS - 1) * SL
      for sz in _sizes(SL_LAST):
        pltpu.sync_copy(acc.at[pl.ds(off, sz)], stage.at[pl.ds(0, sz)])
        pltpu.sync_copy(stage.at[pl.ds(0, sz)],
                        out_hbm.at[pl.ds(c * N + off, sz)])
        off += sz

  return edge_pass


BL = 4096


def _prep_body(deg_ref, x_ref, dinv_ref, xs_ref):
  d = deg_ref[0, :] + deg_ref[1, :] + 1.0
  dinv = lax.rsqrt(d)
  dinv_ref[:, :] = dinv[:, None]
  xs_ref[:, :] = x_ref[:, :] * dinv[:, None]


def _make_prep(N):
  return pl.pallas_call(
      _prep_body,
      grid=(pl.cdiv(N, BL),),
      in_specs=[
          pl.BlockSpec((NC, BL), lambda i: (0, i)),
          pl.BlockSpec((BL, L), lambda i: (i, 0)),
      ],
      out_specs=[
          pl.BlockSpec((BL, 1), lambda i: (i, 0)),
          pl.BlockSpec((BL, L), lambda i: (i, 0)),
      ],
      out_shape=[
          jax.ShapeDtypeStruct((N, 1), jnp.float32),
          jax.ShapeDtypeStruct((N, L), jnp.float32),
      ],
  )


def _mid_body(aggp_ref, xs_ref, dinv_ref, w1_ref, b1_ref, w2_ref, h2s_ref):
  dinv = dinv_ref[:, :]
  t = (aggp_ref[0] + aggp_ref[1] + xs_ref[:, :]) * dinv
  h1 = jnp.dot(t, w1_ref[:, :], preferred_element_type=jnp.float32)
  h1 = jnp.maximum(h1 + b1_ref[:, :], 0.0)
  h2 = jnp.dot(h1, w2_ref[:, :], preferred_element_type=jnp.float32)
  h2s_ref[:, :] = h2 * dinv


def _make_mid(N, H1):
  return pl.pallas_call(
      _mid_body,
      grid=(pl.cdiv(N, BL),),
      in_specs=[
          pl.BlockSpec((NC, BL, L), lambda i: (0, i, 0)),
          pl.BlockSpec((BL, L), lambda i: (i, 0)),
          pl.BlockSpec((BL, 1), lambda i: (i, 0)),
          pl.BlockSpec((L, H1), lambda i: (0, 0)),
          pl.BlockSpec((1, H1), lambda i: (0, 0)),
          pl.BlockSpec((H1, L), lambda i: (0, 0)),
      ],
      out_specs=pl.BlockSpec((BL, L), lambda i: (i, 0)),
      out_shape=jax.ShapeDtypeStruct((N, L), jnp.float32),
  )


def _out_body(agg2p_ref, h2s_ref, dinv_ref, b2_ref, o_ref):
  pre = (agg2p_ref[0] + agg2p_ref[1] + h2s_ref[:, :]) * dinv_ref[:, :]
  pre = pre + b2_ref[:, :]
  lane = lax.broadcasted_iota(jnp.int32, pre.shape, 1)
  prem = jnp.where(lane < 7, pre, -1e30)
  m = jnp.max(prem, axis=1, keepdims=True)
  e = jnp.where(lane < 7, jnp.exp(prem - m), 0.0)
  o_ref[:, :] = prem - m - jnp.log(jnp.sum(e, axis=1, keepdims=True))


def _make_out(N):
  return pl.pallas_call(
      _out_body,
      grid=(pl.cdiv(N, BL),),
      in_specs=[
          pl.BlockSpec((NC, BL, L), lambda i: (0, i, 0)),
          pl.BlockSpec((BL, L), lambda i: (i, 0)),
          pl.BlockSpec((BL, 1), lambda i: (i, 0)),
          pl.BlockSpec((1, L), lambda i: (0, 0)),
      ],
      out_specs=pl.BlockSpec((BL, L), lambda i: (i, 0)),
      out_shape=jax.ShapeDtypeStruct((N, L), jnp.float32),
  )


def kernel(x, edge_index, edge_weight, W1, b1, W2, b2):
  N = x.shape[0]
  E = edge_weight.shape[0]
  H1 = W1.shape[1]

  src2 = edge_index[0].astype(jnp.int32).reshape(E // GRP, GRP)
  dst2 = edge_index[1].astype(jnp.int32).reshape(E // GRP, GRP)
  ew = edge_weight.astype(jnp.float32)

  x16 = jnp.pad(x, ((0, 0), (0, L - x.shape[1])))
  W1p = jnp.pad(W1, ((0, L - W1.shape[0]), (0, 0)))
  W2p = jnp.pad(W2, ((0, 0), (0, L - W2.shape[1])))
  b1p = b1[None, :]
  b2p = jnp.pad(b2, (0, L - b2.shape[0]))[None, :]

  deg2 = _make_deg_pass(E, N)(dst2, ew).reshape(NC, N)
  dinv, xs = _make_prep(N)(deg2, x16)
  edge_pass = _make_edge_pass(E, N)
  agg1 = edge_pass(src2, dst2, ew, xs).reshape(NC, N, L)
  h2s = _make_mid(N, H1)(agg1, xs, dinv, W1p, b1p, W2p)
  agg2 = edge_pass(src2, dst2, ew, h2s).reshape(NC, N, L)
  out16 = _make_out(N)(agg2, h2s, dinv, b2p)
  return out16[:, :7]


# speedup vs baseline: 104.4352x; 1.1243x over previous
import functools

import jax
import jax.numpy as jnp
from jax import lax
from jax.experimental import pallas as pl
from jax.experimental.pallas import tpu as pltpu
from jax.experimental.pallas import tpu_sc as plsc

NC = 2
NS = 16
NW = NC * NS
L = 16
GRP = 128
CHUNK = 512
NGRP = CHUNK // GRP
MUL_UNROLL = 16


def _tile_rows(n):
  sl = -(-n // NS)
  sl = -(-sl // 8) * 8
  return sl, n - (NS - 1) * sl


def _sizes(n, chunk=CHUNK):
  out = [chunk] * (n // chunk)
  if n % chunk:
    out.append(n % chunk)
  return out


CHUNK_D = 2048
NGRP_D = CHUNK_D // GRP


def _make_deg_pass(E, N):
  nchunks = E // CHUNK_D
  base, rem = divmod(nchunks, NW)
  nstep = -(-nchunks // NW)
  SL, SL_LAST = _tile_rows(N)
  mesh = plsc.VectorSubcoreMesh(core_axis_name="c", subcore_axis_name="s")

  @functools.partial(
      pl.kernel,
      out_type=jax.ShapeDtypeStruct((NC * N,), jnp.float32),
      mesh=mesh,
      scratch_types=[
          pltpu.VMEM_SHARED((N,), jnp.float32),
          pltpu.VMEM((2, NGRP_D, GRP), jnp.int32),
          pltpu.VMEM((2, CHUNK_D), jnp.float32),
          pltpu.VMEM((CHUNK_D,), jnp.float32),
          pltpu.SemaphoreType.DMA,
          pltpu.SemaphoreType.DMA,
          pltpu.SemaphoreType.DMA,
          pltpu.SemaphoreType.DMA,
      ],
  )
  def deg_pass(dst_hbm, ew_hbm, out_hbm, deg_sh, dstv, eww, stagebuf,
               si0, si1, ss0, ss1):
    c = lax.axis_index("c")
    s = lax.axis_index("s")
    wid = s * NC + c
    sem_i, sem_s = [si0, si1], [ss0, ss1]
    stage = stagebuf

    def zf(i, carry):
      stage[pl.ds(i * L, L)] = jnp.zeros((L,), jnp.float32)
      return carry
    lax.fori_loop(0, CHUNK_D // L, zf, 0)

    @pl.when(s < NS - 1)
    def _():
      off = 0
      for sz in _sizes(SL, CHUNK_D):
        pltpu.sync_copy(stage.at[pl.ds(0, sz)], deg_sh.at[pl.ds(s * SL + off, sz)])
        off += sz

    @pl.when(s == NS - 1)
    def _():
      off = (NS - 1) * SL
      for sz in _sizes(SL_LAST, CHUNK_D):
        pltpu.sync_copy(stage.at[pl.ds(0, sz)], deg_sh.at[pl.ds(off, sz)])
        off += sz

    plsc.subcore_barrier()

    n_i = base + jnp.where(wid < rem, 1, 0)

    def idx_issue(j, b):
      @pl.when(j < n_i)
      def _():
        cid = wid + NW * j
        pltpu.async_copy(dst_hbm.at[pl.ds(cid * NGRP_D, NGRP_D)], dstv.at[b],
                         sem_i[b])
        pltpu.async_copy(ew_hbm.at[pl.ds(cid * CHUNK_D, CHUNK_D)], eww.at[b],
                         sem_i[b])

    def scatter_issue(j, b):
      @pl.when(j < n_i)
      def _():
        cid = wid + NW * j
        pltpu.make_async_copy(dst_hbm.at[pl.ds(cid * NGRP_D, NGRP_D)],
                              dstv.at[b], sem_i[b]).wait()
        pltpu.make_async_copy(ew_hbm.at[pl.ds(cid * CHUNK_D, CHUNK_D)],
                              eww.at[b], sem_i[b]).wait()
        for g in range(NGRP_D):
          pltpu.async_copy(eww.at[b, pl.ds(g * GRP, GRP)],
                           deg_sh.at[dstv.at[b, g]], sem_s[b], add=True)

    def scatter_drain(j, b):
      @pl.when(j < n_i)
      def _():
        for g in range(NGRP_D):
          pltpu.make_async_copy(eww.at[b, pl.ds(g * GRP, GRP)],
                                deg_sh.at[dstv.at[b, g]], sem_s[b]).wait()

    idx_issue(0, 0)
    idx_issue(1, 1)

    def pipe_body(q, carry):
      j0 = 2 * q
      scatter_issue(j0, 0)
      scatter_drain(j0, 0)
      idx_issue(j0 + 2, 0)
      scatter_issue(j0 + 1, 1)
      scatter_drain(j0 + 1, 1)
      idx_issue(j0 + 3, 1)
      return carry
    lax.fori_loop(0, (nstep + 1) // 2, pipe_body, 0)

    plsc.subcore_barrier()

    @pl.when(s < NS - 1)
    def _():
      off = 0
      for sz in _sizes(SL, CHUNK_D):
        pltpu.sync_copy(deg_sh.at[pl.ds(s * SL + off, sz)], stage.at[pl.ds(0, sz)])
        pltpu.sync_copy(stage.at[pl.ds(0, sz)],
                        out_hbm.at[pl.ds(c * N + s * SL + off, sz)])
        off += sz

    @pl.when(s == NS - 1)
    def _():
      off = (NS - 1) * SL
      for sz in _sizes(SL_LAST, CHUNK_D):
        pltpu.sync_copy(deg_sh.at[pl.ds(off, sz)], stage.at[pl.ds(0, sz)])
        pltpu.sync_copy(stage.at[pl.ds(0, sz)],
                        out_hbm.at[pl.ds(c * N + off, sz)])
        off += sz

  return deg_pass


def _make_edge_pass(E, N):
  nchunks = E // CHUNK
  base, rem = divmod(nchunks, NW)
  nstep = -(-nchunks // NW)
  SL, SL_LAST = _tile_rows(N)
  mesh = plsc.VectorSubcoreMesh(core_axis_name="c", subcore_axis_name="s")

  @functools.partial(
      pl.kernel,
      out_type=jax.ShapeDtypeStruct((NC * N, L), jnp.float32),
      mesh=mesh,
      compiler_params=pltpu.CompilerParams(use_tc_tiling_on_sc=False),
      scratch_types=[
          pltpu.VMEM_SHARED((N, L), jnp.float32),
          pltpu.VMEM((3, NGRP, GRP), jnp.int32),
          pltpu.VMEM((3, NGRP, GRP), jnp.int32),
          pltpu.VMEM((3, CHUNK), jnp.float32),
          pltpu.VMEM((3, CHUNK, L), jnp.float32),
          pltpu.SemaphoreType.DMA,
          pltpu.SemaphoreType.DMA,
          pltpu.SemaphoreType.DMA,
          pltpu.SemaphoreType.DMA,
          pltpu.SemaphoreType.DMA,
          pltpu.SemaphoreType.DMA,
          pltpu.SemaphoreType.DMA,
          pltpu.SemaphoreType.DMA,
          pltpu.SemaphoreType.DMA,
      ],
  )
  def edge_pass(src_hbm, dst_hbm, ew_hbm, table_hbm, out_hbm,
                acc, srcv, dstv, eww, rows,
                si0, si1, si2, sg0, sg1, sg2, ss0, ss1, ss2):
    c = lax.axis_index("c")
    s = lax.axis_index("s")
    wid = s * NC + c
    sem_i, sem_g, sem_s = [si0, si1, si2], [sg0, sg1, sg2], [ss0, ss1, ss2]
    stage = rows.at[0]

    def zf(i, carry):
      stage[i, :] = jnp.zeros((L,), jnp.float32)
      return carry
    lax.fori_loop(0, CHUNK, zf, 0)

    @pl.when(s < NS - 1)
    def _():
      off = 0
      for sz in _sizes(SL):
        pltpu.sync_copy(stage.at[pl.ds(0, sz)], acc.at[pl.ds(s * SL + off, sz)])
        off += sz

    @pl.when(s == NS - 1)
    def _():
      off = (NS - 1) * SL
      for sz in _sizes(SL_LAST):
        pltpu.sync_copy(stage.at[pl.ds(0, sz)], acc.at[pl.ds(off, sz)])
        off += sz

    plsc.subcore_barrier()

    n_i = base + jnp.where(wid < rem, 1, 0)

    def _idx_copies(j, b):
      cid = wid + NW * j
      return [
          (src_hbm.at[pl.ds(cid * NGRP, NGRP)], srcv.at[b]),
          (dst_hbm.at[pl.ds(cid * NGRP, NGRP)], dstv.at[b]),
          (ew_hbm.at[pl.ds(cid * CHUNK, CHUNK)], eww.at[b]),
      ]

    def idx_issue(j, b):
      @pl.when(j < n_i)
      def _():
        for src, dst in _idx_copies(j, b):
          pltpu.async_copy(src, dst, sem_i[b])

    def gather_issue(j, b):
      @pl.when(j < n_i)
      def _():
        for src, dst in _idx_copies(j, b):
          pltpu.make_async_copy(src, dst, sem_i[b]).wait()
        for g in range(NGRP):
          pltpu.async_copy(table_hbm.at[srcv.at[b, g]],
                           rows.at[b, pl.ds(g * GRP, GRP)], sem_g[b])

    def mul_chunk(j, b):
      @pl.when(j < n_i)
      def _():
        for g in range(NGRP):
          pltpu.make_async_copy(table_hbm.at[srcv.at[b, g]],
                                rows.at[b, pl.ds(g * GRP, GRP)],
                                sem_g[b]).wait()
        rb = rows.at[b]
        eb = eww.at[b]

        def mul_body(g, carry2):
          ew16 = eb[pl.ds(g * L, L)]
          for i in range(L):
            e = g * L + i
            rb[e, :] = rb[e, :] * ew16[i]
          return carry2
        lax.fori_loop(0, CHUNK // L, mul_body, 0)

    def scatter_issue(j, b):
      @pl.when(j < n_i)
      def _():
        for g in range(NGRP):
          pltpu.async_copy(rows.at[b, pl.ds(g * GRP, GRP)],
                           acc.at[dstv.at[b, g]], sem_s[b], add=True)

    def scatter_drain(j, b):
      @pl.when((j >= 0) & (j < n_i))
      def _():
        for g in range(NGRP):
          pltpu.make_async_copy(rows.at[b, pl.ds(g * GRP, GRP)],
                                acc.at[dstv.at[b, g]], sem_s[b]).wait()

    idx_issue(0, 0)
    idx_issue(1, 1)
    gather_issue(0, 0)

    def pipe_body(q, carry):
      j0 = 3 * q
      for t in range(3):
        j = j0 + t
        b = t
        gather_issue(j + 1, (t + 1) % 3)
        mul_chunk(j, b)
        scatter_drain(j - 1, (t + 2) % 3)
        scatter_issue(j, b)
        idx_issue(j + 2, (t + 2) % 3)
      return carry
    lax.fori_loop(0, (nstep + 3) // 3, pipe_body, 0)

    plsc.subcore_barrier()

    @pl.when(s < NS - 1)
    def _():
      off = 0
      for sz in _sizes(SL):
        pltpu.sync_copy(acc.at[pl.ds(s * SL + off, sz)], stage.at[pl.ds(0, sz)])
        pltpu.sync_copy(stage.at[pl.ds(0, sz)],
                        out_hbm.at[pl.ds(c * N + s * SL + off, sz)])
        off += sz

    @pl.when(s == NS - 1)
    def _():
      off = (NS - 1) * SL
      for sz in _sizes(SL_LAST):
        pltpu.sync_copy(acc.at[pl.ds(off, sz)], stage.at[pl.ds(0, sz)])
        pltpu.sync_copy(stage.at[pl.ds(0, sz)],
                        out_hbm.at[pl.ds(c * N + off, sz)])
        off += sz

  return edge_pass


BL = 4096


def _prep_body(deg_ref, x_ref, dinv_ref, xs_ref):
  d = deg_ref[0, :] + deg_ref[1, :] + 1.0
  dinv = lax.rsqrt(d)
  dinv_ref[:, :] = dinv[:, None]
  xs_ref[:, :] = x_ref[:, :] * dinv[:, None]


def _make_prep(N):
  return pl.pallas_call(
      _prep_body,
      grid=(pl.cdiv(N, BL),),
      in_specs=[
          pl.BlockSpec((NC, BL), lambda i: (0, i)),
          pl.BlockSpec((BL, L), lambda i: (i, 0)),
      ],
      out_specs=[
          pl.BlockSpec((BL, 1), lambda i: (i, 0)),
          pl.BlockSpec((BL, L), lambda i: (i, 0)),
      ],
      out_shape=[
          jax.ShapeDtypeStruct((N, 1), jnp.float32),
          jax.ShapeDtypeStruct((N, L), jnp.float32),
      ],
  )


def _mid_body(aggp_ref, xs_ref, dinv_ref, w1_ref, b1_ref, w2_ref, h2s_ref):
  dinv = dinv_ref[:, :]
  t = (aggp_ref[0] + aggp_ref[1] + xs_ref[:, :]) * dinv
  h1 = jnp.dot(t, w1_ref[:, :], preferred_element_type=jnp.float32)
  h1 = jnp.maximum(h1 + b1_ref[:, :], 0.0)
  h2 = jnp.dot(h1, w2_ref[:, :], preferred_element_type=jnp.float32)
  h2s_ref[:, :] = h2 * dinv


def _make_mid(N, H1):
  return pl.pallas_call(
      _mid_body,
      grid=(pl.cdiv(N, BL),),
      in_specs=[
          pl.BlockSpec((NC, BL, L), lambda i: (0, i, 0)),
          pl.BlockSpec((BL, L), lambda i: (i, 0)),
          pl.BlockSpec((BL, 1), lambda i: (i, 0)),
          pl.BlockSpec((L, H1), lambda i: (0, 0)),
          pl.BlockSpec((1, H1), lambda i: (0, 0)),
          pl.BlockSpec((H1, L), lambda i: (0, 0)),
      ],
      out_specs=pl.BlockSpec((BL, L), lambda i: (i, 0)),
      out_shape=jax.ShapeDtypeStruct((N, L), jnp.float32),
  )


def _out_body(agg2p_ref, h2s_ref, dinv_ref, b2_ref, o_ref):
  pre = (agg2p_ref[0] + agg2p_ref[1] + h2s_ref[:, :]) * dinv_ref[:, :]
  pre = pre + b2_ref[:, :]
  lane = lax.broadcasted_iota(jnp.int32, pre.shape, 1)
  prem = jnp.where(lane < 7, pre, -1e30)
  m = jnp.max(prem, axis=1, keepdims=True)
  e = jnp.where(lane < 7, jnp.exp(prem - m), 0.0)
  o_ref[:, :] = prem - m - jnp.log(jnp.sum(e, axis=1, keepdims=True))


def _make_out(N):
  return pl.pallas_call(
      _out_body,
      grid=(pl.cdiv(N, BL),),
      in_specs=[
          pl.BlockSpec((NC, BL, L), lambda i: (0, i, 0)),
          pl.BlockSpec((BL, L), lambda i: (i, 0)),
          pl.BlockSpec((BL, 1), lambda i: (i, 0)),
          pl.BlockSpec((1, L), lambda i: (0, 0)),
      ],
      out_specs=pl.BlockSpec((BL, L), lambda i: (i, 0)),
      out_shape=jax.ShapeDtypeStruct((N, L), jnp.float32),
  )


def kernel(x, edge_index, edge_weight, W1, b1, W2, b2):
  N = x.shape[0]
  E = edge_weight.shape[0]
  H1 = W1.shape[1]

  src2 = edge_index[0].astype(jnp.int32).reshape(E // GRP, GRP)
  dst2 = edge_index[1].astype(jnp.int32).reshape(E // GRP, GRP)
  ew = edge_weight.astype(jnp.float32)

  x16 = jnp.pad(x, ((0, 0), (0, L - x.shape[1])))
  W1p = jnp.pad(W1, ((0, L - W1.shape[0]), (0, 0)))
  W2p = jnp.pad(W2, ((0, 0), (0, L - W2.shape[1])))
  b1p = b1[None, :]
  b2p = jnp.pad(b2, (0, L - b2.shape[0]))[None, :]

  deg2 = _make_deg_pass(E, N)(dst2, ew).reshape(NC, N)
  dinv, xs = _make_prep(N)(deg2, x16)
  edge_pass = _make_edge_pass(E, N)
  agg1 = edge_pass(src2, dst2, ew, xs).reshape(NC, N, L)
  h2s = _make_mid(N, H1)(agg1, xs, dinv, W1p, b1p, W2p)
  agg2 = edge_pass(src2, dst2, ew, h2s).reshape(NC, N, L)
  out16 = _make_out(N)(agg2, h2s, dinv, b2p)
  return out16[:, :7]


# speedup vs baseline: 104.9570x; 1.0050x over previous
import functools

import jax
import jax.numpy as jnp
from jax import lax
from jax.experimental import pallas as pl
from jax.experimental.pallas import tpu as pltpu
from jax.experimental.pallas import tpu_sc as plsc

NC = 2
NS = 16
NW = NC * NS
L = 16
GRP = 128
CHUNK = 512
NGRP = CHUNK // GRP
MUL_UNROLL = 16


def _tile_rows(n):
  sl = -(-n // NS)
  sl = -(-sl // 8) * 8
  return sl, n - (NS - 1) * sl


def _sizes(n, chunk=CHUNK):
  out = [chunk] * (n // chunk)
  if n % chunk:
    out.append(n % chunk)
  return out


CHUNK_D = 2048
NGRP_D = CHUNK_D // GRP


def _make_deg_pass(E, N):
  nchunks = E // CHUNK_D
  base, rem = divmod(nchunks, NW)
  nstep = -(-nchunks // NW)
  SL, SL_LAST = _tile_rows(N)
  mesh = plsc.VectorSubcoreMesh(core_axis_name="c", subcore_axis_name="s")

  @functools.partial(
      pl.kernel,
      out_type=jax.ShapeDtypeStruct((NC * N,), jnp.float32),
      mesh=mesh,
      scratch_types=[
          pltpu.VMEM_SHARED((N,), jnp.float32),
          pltpu.VMEM((2, NGRP_D, GRP), jnp.int32),
          pltpu.VMEM((2, CHUNK_D), jnp.float32),
          pltpu.VMEM((CHUNK_D,), jnp.float32),
          pltpu.SemaphoreType.DMA,
          pltpu.SemaphoreType.DMA,
          pltpu.SemaphoreType.DMA,
          pltpu.SemaphoreType.DMA,
      ],
  )
  def deg_pass(dst_hbm, ew_hbm, out_hbm, deg_sh, dstv, eww, stagebuf,
               si0, si1, ss0, ss1):
    c = lax.axis_index("c")
    s = lax.axis_index("s")
    wid = s * NC + c
    sem_i, sem_s = [si0, si1], [ss0, ss1]
    stage = stagebuf

    def zf(i, carry):
      stage[pl.ds(i * L, L)] = jnp.zeros((L,), jnp.float32)
      return carry
    lax.fori_loop(0, CHUNK_D // L, zf, 0)

    @pl.when(s < NS - 1)
    def _():
      off = 0
      for sz in _sizes(SL, CHUNK_D):
        pltpu.sync_copy(stage.at[pl.ds(0, sz)], deg_sh.at[pl.ds(s * SL + off, sz)])
        off += sz

    @pl.when(s == NS - 1)
    def _():
      off = (NS - 1) * SL
      for sz in _sizes(SL_LAST, CHUNK_D):
        pltpu.sync_copy(stage.at[pl.ds(0, sz)], deg_sh.at[pl.ds(off, sz)])
        off += sz

    plsc.subcore_barrier()

    n_i = base + jnp.where(wid < rem, 1, 0)

    def idx_issue(j, b):
      @pl.when(j < n_i)
      def _():
        cid = wid + NW * j
        pltpu.async_copy(dst_hbm.at[pl.ds(cid * NGRP_D, NGRP_D)], dstv.at[b],
                         sem_i[b])
        pltpu.async_copy(ew_hbm.at[pl.ds(cid * CHUNK_D, CHUNK_D)], eww.at[b],
                         sem_i[b])

    def scatter_issue(j, b):
      @pl.when(j < n_i)
      def _():
        cid = wid + NW * j
        pltpu.make_async_copy(dst_hbm.at[pl.ds(cid * NGRP_D, NGRP_D)],
                              dstv.at[b], sem_i[b]).wait()
        pltpu.make_async_copy(ew_hbm.at[pl.ds(cid * CHUNK_D, CHUNK_D)],
                              eww.at[b], sem_i[b]).wait()
        for g in range(NGRP_D):
          pltpu.async_copy(eww.at[b, pl.ds(g * GRP, GRP)],
                           deg_sh.at[dstv.at[b, g]], sem_s[b], add=True)

    def scatter_drain(j, b):
      @pl.when(j < n_i)
      def _():
        for g in range(NGRP_D):
          pltpu.make_async_copy(eww.at[b, pl.ds(g * GRP, GRP)],
                                deg_sh.at[dstv.at[b, g]], sem_s[b]).wait()

    idx_issue(0, 0)
    idx_issue(1, 1)

    def pipe_body(q, carry):
      j0 = 2 * q
      scatter_issue(j0, 0)
      scatter_drain(j0, 0)
      idx_issue(j0 + 2, 0)
      scatter_issue(j0 + 1, 1)
      scatter_drain(j0 + 1, 1)
      idx_issue(j0 + 3, 1)
      return carry
    lax.fori_loop(0, (nstep + 1) // 2, pipe_body, 0)

    plsc.subcore_barrier()

    @pl.when(s < NS - 1)
    def _():
      off = 0
      for sz in _sizes(SL, CHUNK_D):
        pltpu.sync_copy(deg_sh.at[pl.ds(s * SL + off, sz)], stage.at[pl.ds(0, sz)])
        pltpu.sync_copy(stage.at[pl.ds(0, sz)],
                        out_hbm.at[pl.ds(c * N + s * SL + off, sz)])
        off += sz

    @pl.when(s == NS - 1)
    def _():
      off = (NS - 1) * SL
      for sz in _sizes(SL_LAST, CHUNK_D):
        pltpu.sync_copy(deg_sh.at[pl.ds(off, sz)], stage.at[pl.ds(0, sz)])
        pltpu.sync_copy(stage.at[pl.ds(0, sz)],
                        out_hbm.at[pl.ds(c * N + off, sz)])
        off += sz

  return deg_pass


def _make_edge_pass(E, N):
  nchunks = E // CHUNK
  base, rem = divmod(nchunks, NW)
  nstep = -(-nchunks // NW)
  SL, SL_LAST = _tile_rows(N)
  mesh = plsc.VectorSubcoreMesh(core_axis_name="c", subcore_axis_name="s")

  @functools.partial(
      pl.kernel,
      out_type=jax.ShapeDtypeStruct((NC * N, L), jnp.float32),
      mesh=mesh,
      compiler_params=pltpu.CompilerParams(use_tc_tiling_on_sc=False),
      scratch_types=[
          pltpu.VMEM_SHARED((N, L), jnp.float32),
          pltpu.VMEM((3, NGRP, GRP), jnp.int32),
          pltpu.VMEM((3, NGRP, GRP), jnp.int32),
          pltpu.VMEM((3, CHUNK), jnp.float32),
          pltpu.VMEM((3, CHUNK, L), jnp.float32),
          pltpu.SemaphoreType.DMA,
          pltpu.SemaphoreType.DMA,
          pltpu.SemaphoreType.DMA,
          pltpu.SemaphoreType.DMA,
          pltpu.SemaphoreType.DMA,
          pltpu.SemaphoreType.DMA,
          pltpu.SemaphoreType.DMA,
          pltpu.SemaphoreType.DMA,
          pltpu.SemaphoreType.DMA,
      ],
  )
  def edge_pass(src_hbm, dst_hbm, ew_hbm, table_hbm, out_hbm,
                acc, srcv, dstv, eww, rows,
                si0, si1, si2, sg0, sg1, sg2, ss0, ss1, ss2):
    c = lax.axis_index("c")
    s = lax.axis_index("s")
    wid = s * NC + c
    sem_i, sem_g, sem_s = [si0, si1, si2], [sg0, sg1, sg2], [ss0, ss1, ss2]
    stage = rows.at[0]

    def zf(i, carry):
      stage[i, :] = jnp.zeros((L,), jnp.float32)
      return carry
    lax.fori_loop(0, CHUNK, zf, 0)

    def _zero_blocks(start, total):
      copies = []
      off = 0
      for sz in _sizes(total):
        copies.append((stage.at[pl.ds(0, sz)], acc.at[pl.ds(start + off, sz)]))
        off += sz
      for src, dst in copies:
        pltpu.async_copy(src, dst, sem_s[0])
      for src, dst in copies:
        pltpu.make_async_copy(src, dst, sem_s[0]).wait()

    @pl.when(s < NS - 1)
    def _():
      _zero_blocks(s * SL, SL)

    @pl.when(s == NS - 1)
    def _():
      _zero_blocks((NS - 1) * SL, SL_LAST)

    plsc.subcore_barrier()

    n_i = base + jnp.where(wid < rem, 1, 0)

    def _idx_copies(j, b):
      cid = wid + NW * j
      return [
          (src_hbm.at[pl.ds(cid * NGRP, NGRP)], srcv.at[b]),
          (dst_hbm.at[pl.ds(cid * NGRP, NGRP)], dstv.at[b]),
          (ew_hbm.at[pl.ds(cid * CHUNK, CHUNK)], eww.at[b]),
      ]

    def idx_issue(j, b):
      @pl.when(j < n_i)
      def _():
        for src, dst in _idx_copies(j, b):
          pltpu.async_copy(src, dst, sem_i[b])

    def gather_issue(j, b):
      @pl.when(j < n_i)
      def _():
        for src, dst in _idx_copies(j, b):
          pltpu.make_async_copy(src, dst, sem_i[b]).wait()
        for g in range(NGRP):
          pltpu.async_copy(table_hbm.at[srcv.at[b, g]],
                           rows.at[b, pl.ds(g * GRP, GRP)], sem_g[b])

    def mul_chunk(j, b):
      @pl.when(j < n_i)
      def _():
        for g in range(NGRP):
          pltpu.make_async_copy(table_hbm.at[srcv.at[b, g]],
                                rows.at[b, pl.ds(g * GRP, GRP)],
                                sem_g[b]).wait()
        rb = rows.at[b]
        eb = eww.at[b]

        def mul_body(g, carry2):
          ew16 = eb[pl.ds(g * L, L)]
          for i in range(L):
            e = g * L + i
            rb[e, :] = rb[e, :] * ew16[i]
          return carry2
        lax.fori_loop(0, CHUNK // L, mul_body, 0)

    def scatter_issue(j, b):
      @pl.when(j < n_i)
      def _():
        for g in range(NGRP):
          pltpu.async_copy(rows.at[b, pl.ds(g * GRP, GRP)],
                           acc.at[dstv.at[b, g]], sem_s[b], add=True)

    def scatter_drain(j, b):
      @pl.when((j >= 0) & (j < n_i))
      def _():
        for g in range(NGRP):
          pltpu.make_async_copy(rows.at[b, pl.ds(g * GRP, GRP)],
                                acc.at[dstv.at[b, g]], sem_s[b]).wait()

    idx_issue(0, 0)
    idx_issue(1, 1)
    gather_issue(0, 0)

    def pipe_body(q, carry):
      j0 = 3 * q
      for t in range(3):
        j = j0 + t
        b = t
        gather_issue(j + 1, (t + 1) % 3)
        mul_chunk(j, b)
        scatter_drain(j - 1, (t + 2) % 3)
        scatter_issue(j, b)
        idx_issue(j + 2, (t + 2) % 3)
      return carry
    lax.fori_loop(0, (nstep + 3) // 3, pipe_body, 0)

    plsc.subcore_barrier()

    def _copy_out(start, total):
      blocks = []
      off = 0
      for sz in _sizes(total):
        blocks.append((off, sz))
        off += sz
      for i, (boff, sz) in enumerate(blocks):
        b = i % 3
        if i >= 3:
          poff, psz = blocks[i - 3]
          pltpu.make_async_copy(
              rows.at[b, pl.ds(0, psz)],
              out_hbm.at[pl.ds(c * N + start + poff, psz)], sem_s[b]).wait()
        pltpu.sync_copy(acc.at[pl.ds(start + boff, sz)],
                        rows.at[b, pl.ds(0, sz)])
        pltpu.async_copy(rows.at[b, pl.ds(0, sz)],
                         out_hbm.at[pl.ds(c * N + start + boff, sz)], sem_s[b])
      for i in range(max(0, len(blocks) - 3), len(blocks)):
        boff, sz = blocks[i]
        pltpu.make_async_copy(
            rows.at[i % 3, pl.ds(0, sz)],
            out_hbm.at[pl.ds(c * N + start + boff, sz)], sem_s[i % 3]).wait()

    @pl.when(s < NS - 1)
    def _():
      _copy_out(s * SL, SL)

    @pl.when(s == NS - 1)
    def _():
      _copy_out((NS - 1) * SL, SL_LAST)

  return edge_pass


BL = 4096


def _prep_body(deg_ref, x_ref, dinv_ref, xs_ref):
  d = deg_ref[0, :] + deg_ref[1, :] + 1.0
  dinv = lax.rsqrt(d)
  dinv_ref[:, :] = dinv[:, None]
  xs_ref[:, :] = x_ref[:, :] * dinv[:, None]


def _make_prep(N):
  return pl.pallas_call(
      _prep_body,
      grid=(pl.cdiv(N, BL),),
      in_specs=[
          pl.BlockSpec((NC, BL), lambda i: (0, i)),
          pl.BlockSpec((BL, L), lambda i: (i, 0)),
      ],
      out_specs=[
          pl.BlockSpec((BL, 1), lambda i: (i, 0)),
          pl.BlockSpec((BL, L), lambda i: (i, 0)),
      ],
      out_shape=[
          jax.ShapeDtypeStruct((N, 1), jnp.float32),
          jax.ShapeDtypeStruct((N, L), jnp.float32),
      ],
  )


def _mid_body(aggp_ref, xs_ref, dinv_ref, w1_ref, b1_ref, w2_ref, h2s_ref):
  dinv = dinv_ref[:, :]
  t = (aggp_ref[0] + aggp_ref[1] + xs_ref[:, :]) * dinv
  h1 = jnp.dot(t, w1_ref[:, :], preferred_element_type=jnp.float32)
  h1 = jnp.maximum(h1 + b1_ref[:, :], 0.0)
  h2 = jnp.dot(h1, w2_ref[:, :], preferred_element_type=jnp.float32)
  h2s_ref[:, :] = h2 * dinv


def _make_mid(N, H1):
  return pl.pallas_call(
      _mid_body,
      grid=(pl.cdiv(N, BL),),
      in_specs=[
          pl.BlockSpec((NC, BL, L), lambda i: (0, i, 0)),
          pl.BlockSpec((BL, L), lambda i: (i, 0)),
          pl.BlockSpec((BL, 1), lambda i: (i, 0)),
          pl.BlockSpec((L, H1), lambda i: (0, 0)),
          pl.BlockSpec((1, H1), lambda i: (0, 0)),
          pl.BlockSpec((H1, L), lambda i: (0, 0)),
      ],
      out_specs=pl.BlockSpec((BL, L), lambda i: (i, 0)),
      out_shape=jax.ShapeDtypeStruct((N, L), jnp.float32),
  )


def _out_body(agg2p_ref, h2s_ref, dinv_ref, b2_ref, o_ref):
  pre = (agg2p_ref[0] + agg2p_ref[1] + h2s_ref[:, :]) * dinv_ref[:, :]
  pre = pre + b2_ref[:, :]
  lane = lax.broadcasted_iota(jnp.int32, pre.shape, 1)
  prem = jnp.where(lane < 7, pre, -1e30)
  m = jnp.max(prem, axis=1, keepdims=True)
  e = jnp.where(lane < 7, jnp.exp(prem - m), 0.0)
  o_ref[:, :] = prem - m - jnp.log(jnp.sum(e, axis=1, keepdims=True))


def _make_out(N):
  return pl.pallas_call(
      _out_body,
      grid=(pl.cdiv(N, BL),),
      in_specs=[
          pl.BlockSpec((NC, BL, L), lambda i: (0, i, 0)),
          pl.BlockSpec((BL, L), lambda i: (i, 0)),
          pl.BlockSpec((BL, 1), lambda i: (i, 0)),
          pl.BlockSpec((1, L), lambda i: (0, 0)),
      ],
      out_specs=pl.BlockSpec((BL, L), lambda i: (i, 0)),
      out_shape=jax.ShapeDtypeStruct((N, L), jnp.float32),
  )


def kernel(x, edge_index, edge_weight, W1, b1, W2, b2):
  N = x.shape[0]
  E = edge_weight.shape[0]
  H1 = W1.shape[1]

  src2 = edge_index[0].astype(jnp.int32).reshape(E // GRP, GRP)
  dst2 = edge_index[1].astype(jnp.int32).reshape(E // GRP, GRP)
  ew = edge_weight.astype(jnp.float32)

  x16 = jnp.pad(x, ((0, 0), (0, L - x.shape[1])))
  W1p = jnp.pad(W1, ((0, L - W1.shape[0]), (0, 0)))
  W2p = jnp.pad(W2, ((0, 0), (0, L - W2.shape[1])))
  b1p = b1[None, :]
  b2p = jnp.pad(b2, (0, L - b2.shape[0]))[None, :]

  deg2 = _make_deg_pass(E, N)(dst2, ew).reshape(NC, N)
  dinv, xs = _make_prep(N)(deg2, x16)
  edge_pass = _make_edge_pass(E, N)
  agg1 = edge_pass(src2, dst2, ew, xs).reshape(NC, N, L)
  h2s = _make_mid(N, H1)(agg1, xs, dinv, W1p, b1p, W2p)
  agg2 = edge_pass(src2, dst2, ew, h2s).reshape(NC, N, L)
  out16 = _make_out(N)(agg2, h2s, dinv, b2p)
  return out16[:, :7]
